# Initial kernel scaffold; baseline (speedup 1.0000x reference)
import jax
import jax.numpy as jnp
from jax.experimental import pallas as pl


def kernel(node_features, edge_index, node_positions, node_radii, W1, b1, W2, b2, W3, b3):
    raise NotImplementedError("write your pallas kernel here")


# speedup vs baseline: 4.9940x; 4.9940x over previous
import functools

import jax
import jax.numpy as jnp
import numpy as np
from jax import lax
from jax.experimental import pallas as pl
from jax.experimental.pallas import tpu as pltpu
from jax.experimental.pallas import tpu_sc as plsc

N_NODES = 10000
N_EDGES = 320000
FEAT = 128

_NC, _NS, _L = 2, 16, 16
_NW = _NC * _NS
_CH = 128
_NCHUNKS = N_EDGES // _CH


def _sc_scatter_body(src_hbm, dst_hbm, a_hbm, src_v, dst_v, idx1_v, idx2_v,
                     ones_v, sem):
    wid = lax.axis_index("s") * _NC + lax.axis_index("c")
    for v in range(_CH // _L):
        ones_v[pl.ds(v * _L, _L)] = jnp.full((_L,), 1.0, jnp.float32)

    base_chunks = _NCHUNKS // _NW
    n_my = base_chunks + jnp.where(wid < (_NCHUNKS % _NW), 1, 0)

    def chunk_body(k, carry):
        t = wid + _NW * k
        base = t * _CH
        pltpu.sync_copy(src_hbm.at[pl.ds(base, _CH)], src_v)
        pltpu.sync_copy(dst_hbm.at[pl.ds(base, _CH)], dst_v)
        for v in range(_CH // _L):
            sl = pl.ds(v * _L, _L)
            s = src_v[sl]
            d = dst_v[sl]
            idx1_v[sl] = s * N_NODES + d
            idx2_v[sl] = d * N_NODES + s
        cp1 = pltpu.async_copy(ones_v, a_hbm.at[idx1_v], sem)
        cp2 = pltpu.async_copy(ones_v, a_hbm.at[idx2_v], sem)
        cp1.wait()
        cp2.wait()
        return carry

    lax.fori_loop(0, n_my, chunk_body, 0)


@functools.cache
def _get_sc_scatter():
    return pl.kernel(
        _sc_scatter_body,
        out_type=(),
        mesh=plsc.VectorSubcoreMesh(core_axis_name="c", subcore_axis_name="s"),
        scratch_types=[
            pltpu.VMEM((_CH,), jnp.int32),
            pltpu.VMEM((_CH,), jnp.int32),
            pltpu.VMEM((_CH,), jnp.int32),
            pltpu.VMEM((_CH,), jnp.int32),
            pltpu.VMEM((_CH,), jnp.float32),
            pltpu.SemaphoreType.DMA,
        ],
    )

_BM = 400
_MI = N_NODES // _BM


def _tc_body(a_blk, xaug_blk, x_blk, w1_ref, b1_ref, w2_ref, b2_ref,
             w3_ref, b3_ref, upd_ref, tap_ref, ssum_ref, viol_ref):
    i = pl.program_id(0)

    a16 = a_blk[...].astype(jnp.bfloat16)
    acc = jnp.dot(a16, xaug_blk[...],
                  preferred_element_type=jnp.float32)

    x = x_blk[...]
    nsum = acc[:, :FEAT]
    cnt = acc[:, FEAT:FEAT + 1]
    has_nb = cnt > 0.0
    nmean = nsum / jnp.maximum(cnt, 1.0)
    combined = jnp.concatenate([x, nmean], axis=1)
    h = jnp.maximum(
        jnp.dot(combined, w1_ref[...],
                preferred_element_type=jnp.float32) + b1_ref[...], 0.0)
    h = jnp.maximum(
        jnp.dot(h, w2_ref[...],
                preferred_element_type=jnp.float32) + b2_ref[...], 0.0)
    logits = jnp.sum(h * w3_ref[...], axis=1, keepdims=True) + b3_ref[...]
    score = jax.nn.sigmoid(logits)
    gain = jnp.where(has_nb, 0.05 * score, 0.0)
    upd_ref[...] = x + gain * jnp.tanh(x)
    tap = jnp.where(has_nb, score, 1.0)
    tap_ref[...] = tap
    part_sum = jnp.sum(tap).reshape(1, 1)
    part_viol = jnp.sum((tap < 0.7).astype(jnp.int32)).reshape(1, 1)

    @pl.when(i == 0)
    def _first():
        ssum_ref[...] = part_sum
        viol_ref[...] = part_viol

    @pl.when(i > 0)
    def _rest():
        ssum_ref[...] += part_sum
        viol_ref[...] += part_viol


_tc_fused = pl.pallas_call(
    _tc_body,
    grid=(_MI,),
    in_specs=[
        pl.BlockSpec((_BM, N_NODES), lambda i: (i, 0)),
        pl.BlockSpec((N_NODES, 2 * FEAT), lambda i: (0, 0)),
        pl.BlockSpec((_BM, FEAT), lambda i: (i, 0)),
        pl.BlockSpec((2 * FEAT, 64), lambda i: (0, 0)),
        pl.BlockSpec((1, 64), lambda i: (0, 0)),
        pl.BlockSpec((64, 32), lambda i: (0, 0)),
        pl.BlockSpec((1, 32), lambda i: (0, 0)),
        pl.BlockSpec((1, 32), lambda i: (0, 0)),
        pl.BlockSpec((1, 1), lambda i: (0, 0)),
    ],
    out_specs=[
        pl.BlockSpec((_BM, FEAT), lambda i: (i, 0)),
        pl.BlockSpec((_BM, 1), lambda i: (i, 0)),
        pl.BlockSpec((1, 1), lambda i: (0, 0)),
        pl.BlockSpec((1, 1), lambda i: (0, 0)),
    ],
    out_shape=[
        jax.ShapeDtypeStruct((N_NODES, FEAT), jnp.float32),
        jax.ShapeDtypeStruct((N_NODES, 1), jnp.float32),
        jax.ShapeDtypeStruct((1, 1), jnp.float32),
        jax.ShapeDtypeStruct((1, 1), jnp.int32),
    ],
    compiler_params=pltpu.CompilerParams(
        dimension_semantics=("arbitrary",)),
)


def kernel(node_features, edge_index, node_positions, node_radii,
           W1, b1, W2, b2, W3, b3):
    del node_positions, node_radii
    src = edge_index[0].astype(jnp.int32)
    dst = edge_index[1].astype(jnp.int32)

    a_ref = jax.new_ref(jnp.zeros((N_NODES * N_NODES,), jnp.float32))
    _get_sc_scatter()(src, dst, a_ref)
    A = a_ref[...].reshape(N_NODES, N_NODES)

    xaug = jnp.concatenate(
        [node_features,
         jnp.ones((N_NODES, 1), jnp.float32),
         jnp.zeros((N_NODES, FEAT - 1), jnp.float32)], axis=1
    ).astype(jnp.bfloat16)

    updated, tap, ssum, viol = _tc_fused(
        A, xaug, node_features,
        W1, b1.reshape(1, 64), W2, b2.reshape(1, 32),
        W3.reshape(1, 32), b3.reshape(1, 1))

    tapering_scores = tap[:, 0]
    avg_consistency = ssum[0, 0] / np.float32(N_NODES)
    num_violations = viol[0, 0]
    return updated, tapering_scores, avg_consistency, num_violations
